# baseline (speedup 1.0000x reference)
import functools

import jax
import jax.numpy as jnp
from jax import lax
from jax.experimental import pallas as pl
from jax.experimental.pallas import tpu as pltpu
from jax.experimental.pallas import tpu_sc as plsc

D0 = 8192
D1 = 4096
TOTAL = D0 * D1
NWORK = 32
PER_W = TOTAL // NWORK
CHUNK = 32768
NCHUNK = PER_W // CHUNK
LANES = 16

_mesh = plsc.VectorSubcoreMesh(core_axis_name="c", subcore_axis_name="s")


@functools.partial(
    pl.kernel,
    mesh=_mesh,
    out_type=[
        jax.ShapeDtypeStruct((NWORK, LANES), jnp.float32),
        jax.ShapeDtypeStruct((NWORK, LANES), jnp.int32),
    ],
    scratch_types=[
        pltpu.VMEM((CHUNK,), jnp.float32),
        pltpu.VMEM((LANES,), jnp.float32),
        pltpu.VMEM((LANES,), jnp.int32),
    ],
)
def _sc_scan(x_hbm, val_out, idx_out, buf, vres, ires):
    cid = lax.axis_index("c")
    sid = lax.axis_index("s")
    wid = sid * 2 + cid
    base = wid * PER_W
    lane = lax.iota(jnp.int32, LANES)

    m0 = jnp.full((LANES,), -jnp.inf, jnp.float32)
    i0 = jnp.zeros((LANES,), jnp.int32)

    def chunk_body(c, carry):
        m, i = carry
        start = base + c * CHUNK
        pltpu.sync_copy(x_hbm.at[pl.ds(start, CHUNK)], buf)

        def vec_body(j, carry2):
            mv, iv = carry2
            v = buf[pl.ds(j * LANES, LANES)]
            idxv = (start + j * LANES) + lane
            take = v > mv
            return jnp.where(take, v, mv), jnp.where(take, idxv, iv)

        return lax.fori_loop(0, CHUNK // LANES, vec_body, (m, i))

    m, i = lax.fori_loop(0, NCHUNK, chunk_body, (m0, i0))
    vres[...] = m
    ires[...] = i
    pltpu.sync_copy(vres, val_out.at[wid])
    pltpu.sync_copy(ires, idx_out.at[wid])


def _finish_body(val_ref, idx_ref, out_ref):
    v = val_ref[...]
    i = idx_ref[...]
    m = jnp.max(v)
    cand = jnp.where(v == m, i, jnp.int32(2147483647))
    best = jnp.min(cand)
    out_ref[0] = best // D0
    out_ref[1] = best % D1


_tc_finish = pl.pallas_call(
    _finish_body,
    out_shape=jax.ShapeDtypeStruct((2,), jnp.int32),
    out_specs=pl.BlockSpec(memory_space=pltpu.SMEM),
)


def kernel(inputs):
    x = jnp.reshape(inputs, (TOTAL,))
    vals, idxs = _sc_scan(x)
    return _tc_finish(
        jnp.reshape(vals, (4, 128)), jnp.reshape(idxs, (4, 128))
    )
